# baseline (speedup 1.0000x reference)
import functools

import jax
import jax.numpy as jnp
from jax import lax
from jax.experimental import pallas as pl
from jax.experimental.pallas import tpu as pltpu
from jax.experimental.pallas import tpu_sc as plsc

_B = 1024
_K = 16
_M = 112
_NCAND = 128
_D = 64 + _NCAND * _NCAND
_BL = 1024
_JCH = 4096
_NJQ = (_D - 64) // _JCH
_NBB = _B // _BL
_NW = 32
_BPW = _B // _NW
_NEG = -1e30


def _sweep_body(xt_ref, tail_ref, ct_ref, dt_ref, acc_out, d_out,
                bias_s, acc_s):
    jq = pl.program_id(1)

    @pl.when(jq == 0)
    def _build():
        pos = jnp.full((8, _BL), -_NEG, jnp.float32)
        negv = jnp.full((8, _BL), _NEG, jnp.float32)
        for g in range(_NCAND // 8):
            ci = lax.broadcasted_iota(jnp.int32, (8, _BL), 0) + 8 * g
            b = negv
            for k in range(_K):
                ck = jnp.broadcast_to(ct_ref[pl.ds(k, 1), :], (8, _BL))
                b = jnp.where(ci == ck, pos, b)
            bias_s[pl.ds(8 * g, 8), :] = b
        acc_s[...] = jnp.full((_NCAND, _BL), _NEG, jnp.float32)

    def accum(sub, cidx, dhalf):
        safe = jnp.maximum(cidx, 0)
        brow = jnp.broadcast_to(bias_s[pl.ds(safe, 1), :], (64, _BL))
        brow = jnp.where(cidx >= 0, brow, _NEG)
        reg = acc_s[pl.ds(64 * dhalf, 64), :]
        acc_s[pl.ds(64 * dhalf, 64), :] = jnp.maximum(
            reg, jnp.minimum(sub, brow))

    for s in range(_JCH // 64):
        sub = xt_ref[pl.ds(64 * s, 64), :]
        toff = (s - 1) // 2 if s > 0 else -1
        accum(sub, 32 * jq + toff, 1 if s % 2 == 0 else 0)

    @pl.when(jq == _NJQ - 1)
    def _finish():
        accum(tail_ref[...], jnp.int32(_NCAND - 1), 1)
        acc_out[...] = jnp.transpose(acc_s[...])
        pad = jnp.zeros((_NCAND - _M, _BL), jnp.int32)
        d_out[...] = jnp.transpose(
            jnp.concatenate([dt_ref[...], pad], axis=0))


_tc_sweep = pl.pallas_call(
    _sweep_body,
    grid=(_NBB, _NJQ),
    in_specs=[
        pl.BlockSpec((_JCH, _BL), lambda bb, jq: (jq, bb)),
        pl.BlockSpec((64, _BL), lambda bb, jq: ((_D - 64) // 64, bb)),
        pl.BlockSpec((_K, _BL), lambda bb, jq: (0, bb)),
        pl.BlockSpec((_M, _BL), lambda bb, jq: (0, bb)),
    ],
    out_specs=[
        pl.BlockSpec((_BL, _NCAND), lambda bb, jq: (bb, 0)),
        pl.BlockSpec((_BL, _NCAND), lambda bb, jq: (bb, 0)),
    ],
    out_shape=[
        jax.ShapeDtypeStruct((_B, _NCAND), jnp.float32),
        jax.ShapeDtypeStruct((_B, _NCAND), jnp.int32),
    ],
    scratch_shapes=[
        pltpu.VMEM((_NCAND, _BL), jnp.float32),
        pltpu.VMEM((_NCAND, _BL), jnp.float32),
    ],
    compiler_params=pltpu.CompilerParams(
        dimension_semantics=("arbitrary", "arbitrary")),
)


def _sc_body(acc_hbm, d_hbm, thr_hbm, out_hbm,
             acc_v, d_v, thr_v, stage, sem):
    cid = lax.axis_index("c")
    sid = lax.axis_index("s")
    wid = sid * 2 + cid
    base = wid * _BPW

    for i in range(_BPW):
        pltpu.async_copy(acc_hbm.at[base + i],
                         acc_v.at[pl.ds(i * _NCAND, _NCAND)], sem)
        pltpu.async_copy(d_hbm.at[base + i],
                         d_v.at[pl.ds(i * _NCAND, _NCAND)], sem)
    for _ in range(2 * _BPW):
        pltpu.make_async_copy(acc_hbm.at[0],
                              acc_v.at[pl.ds(0, _NCAND)], sem).wait()
    pltpu.sync_copy(thr_hbm, thr_v)

    lanes = lax.iota(jnp.int32, 16)
    thr = thr_v[...]

    def row_body(i, acc):
        mvec = jnp.full((16,), _NEG, jnp.float32)
        for j in range(_M // 16):
            dj = d_v[pl.ds(pl.multiple_of(i * _NCAND + j * 16, 8), 16)]
            g = plsc.load_gather(acc_v, [dj + i * _NCAND])
            mvec = jnp.maximum(mvec, g)
        rowmax = plsc.cummax(mvec)
        sig = 1.0 / (1.0 + jnp.exp((thr - rowmax) * 0.1))
        return acc + jnp.where(lanes == 15, sig, 0.0)

    acc = lax.fori_loop(0, _BPW, row_body, jnp.zeros((16,), jnp.float32))
    stage[...] = acc
    pltpu.sync_copy(stage, out_hbm.at[pl.ds(pl.multiple_of(wid * 16, 8), 16)])


@functools.cache
def _sc_final_kernel():
    return functools.partial(
        pl.kernel,
        out_type=jax.ShapeDtypeStruct((_NW * 16,), jnp.float32),
        mesh=plsc.VectorSubcoreMesh(core_axis_name="c", subcore_axis_name="s"),
        compiler_params=pltpu.CompilerParams(
            needs_layout_passes=False, use_tc_tiling_on_sc=True),
        scratch_types=[
            pltpu.VMEM((_BPW * _NCAND,), jnp.float32),
            pltpu.VMEM((_BPW * _NCAND,), jnp.int32),
            pltpu.VMEM((16,), jnp.float32),
            pltpu.VMEM((16,), jnp.float32),
            pltpu.SemaphoreType.DMA,
        ],
    )(_sc_body)


def kernel(c_indices, d_indices, input, n_voters, num_winners, batch_size,
           num_candidates):
    xt = jnp.transpose(input)
    ct = jnp.transpose(c_indices.astype(jnp.int32))
    dt = jnp.transpose(d_indices.astype(jnp.int32))
    thr = jnp.full((16,), (n_voters // 2 + 1), dtype=jnp.float32)
    acc, dpk = _tc_sweep(xt, xt, ct, dt)
    partials = _sc_final_kernel()(acc, dpk, thr)
    return jnp.sum(partials)


# speedup vs baseline: 1.0567x; 1.0567x over previous
import functools

import jax
import jax.numpy as jnp
from jax import lax
from jax.experimental import pallas as pl
from jax.experimental.pallas import tpu as pltpu
from jax.experimental.pallas import tpu_sc as plsc

_B = 1024
_K = 16
_M = 112
_NCAND = 128
_D = 64 + _NCAND * _NCAND
_BL = 512
_JCH = 4096
_NJQ = (_D - 64) // _JCH
_NBB = _B // _BL
_NW = 32
_BPW = _B // _NW
_NEG = -1e30


def _sweep_body(xt_ref, tail_ref, ct_ref, dt_ref, acc_out, d_out,
                bias_s, acc_s):
    jq = pl.program_id(1)

    @pl.when(jq == 0)
    def _build():
        pos = jnp.full((8, _BL), -_NEG, jnp.float32)
        negv = jnp.full((8, _BL), _NEG, jnp.float32)
        for g in range(_NCAND // 8):
            ci = lax.broadcasted_iota(jnp.int32, (8, _BL), 0) + 8 * g
            b = negv
            for k in range(_K):
                ck = jnp.broadcast_to(ct_ref[pl.ds(k, 1), :], (8, _BL))
                b = jnp.where(ci == ck, pos, b)
            bias_s[pl.ds(8 * g, 8), :] = b
        acc_s[...] = jnp.full((_NCAND, _BL), _NEG, jnp.float32)

    def accum(sub, cidx, dhalf):
        safe = jnp.maximum(cidx, 0)
        brow = jnp.broadcast_to(bias_s[pl.ds(safe, 1), :], (64, _BL))
        brow = jnp.where(cidx >= 0, brow, _NEG)
        reg = acc_s[pl.ds(64 * dhalf, 64), :]
        acc_s[pl.ds(64 * dhalf, 64), :] = jnp.maximum(
            reg, jnp.minimum(sub, brow))

    for s in range(_JCH // 64):
        sub = xt_ref[pl.ds(64 * s, 64), :]
        toff = (s - 1) // 2 if s > 0 else -1
        accum(sub, 32 * jq + toff, 1 if s % 2 == 0 else 0)

    @pl.when(jq == _NJQ - 1)
    def _finish():
        accum(tail_ref[...], jnp.int32(_NCAND - 1), 1)
        acc_out[...] = jnp.transpose(acc_s[...])
        pad = jnp.zeros((_NCAND - _M, _BL), jnp.int32)
        d_out[...] = jnp.transpose(
            jnp.concatenate([dt_ref[...], pad], axis=0))


_tc_sweep = pl.pallas_call(
    _sweep_body,
    grid=(_NBB, _NJQ),
    in_specs=[
        pl.BlockSpec((_JCH, _BL), lambda bb, jq: (jq, bb)),
        pl.BlockSpec((64, _BL), lambda bb, jq: ((_D - 64) // 64, bb)),
        pl.BlockSpec((_K, _BL), lambda bb, jq: (0, bb)),
        pl.BlockSpec((_M, _BL), lambda bb, jq: (0, bb)),
    ],
    out_specs=[
        pl.BlockSpec((_BL, _NCAND), lambda bb, jq: (bb, 0)),
        pl.BlockSpec((_BL, _NCAND), lambda bb, jq: (bb, 0)),
    ],
    out_shape=[
        jax.ShapeDtypeStruct((_B, _NCAND), jnp.float32),
        jax.ShapeDtypeStruct((_B, _NCAND), jnp.int32),
    ],
    scratch_shapes=[
        pltpu.VMEM((_NCAND, _BL), jnp.float32),
        pltpu.VMEM((_NCAND, _BL), jnp.float32),
    ],
    compiler_params=pltpu.CompilerParams(
        dimension_semantics=("arbitrary", "arbitrary")),
)


def _sc_body(acc_hbm, d_hbm, thr_hbm, out_hbm,
             acc_v, d_v, thr_v, stage, sem):
    cid = lax.axis_index("c")
    sid = lax.axis_index("s")
    wid = sid * 2 + cid
    base = wid * _BPW

    for i in range(_BPW):
        pltpu.async_copy(acc_hbm.at[base + i],
                         acc_v.at[pl.ds(i * _NCAND, _NCAND)], sem)
        pltpu.async_copy(d_hbm.at[base + i],
                         d_v.at[pl.ds(i * _NCAND, _NCAND)], sem)
    for _ in range(2 * _BPW):
        pltpu.make_async_copy(acc_hbm.at[0],
                              acc_v.at[pl.ds(0, _NCAND)], sem).wait()
    pltpu.sync_copy(thr_hbm, thr_v)

    lanes = lax.iota(jnp.int32, 16)
    thr = thr_v[...]

    def row_body(i, acc):
        mvec = jnp.full((16,), _NEG, jnp.float32)
        for j in range(_M // 16):
            dj = d_v[pl.ds(pl.multiple_of(i * _NCAND + j * 16, 8), 16)]
            g = plsc.load_gather(acc_v, [dj + i * _NCAND])
            mvec = jnp.maximum(mvec, g)
        rowmax = plsc.cummax(mvec)
        sig = 1.0 / (1.0 + jnp.exp((thr - rowmax) * 0.1))
        return acc + jnp.where(lanes == 15, sig, 0.0)

    acc = lax.fori_loop(0, _BPW, row_body, jnp.zeros((16,), jnp.float32))
    stage[...] = acc
    pltpu.sync_copy(stage, out_hbm.at[pl.ds(pl.multiple_of(wid * 16, 8), 16)])


@functools.cache
def _sc_final_kernel():
    return functools.partial(
        pl.kernel,
        out_type=jax.ShapeDtypeStruct((_NW * 16,), jnp.float32),
        mesh=plsc.VectorSubcoreMesh(core_axis_name="c", subcore_axis_name="s"),
        compiler_params=pltpu.CompilerParams(
            needs_layout_passes=False, use_tc_tiling_on_sc=True),
        scratch_types=[
            pltpu.VMEM((_BPW * _NCAND,), jnp.float32),
            pltpu.VMEM((_BPW * _NCAND,), jnp.int32),
            pltpu.VMEM((16,), jnp.float32),
            pltpu.VMEM((16,), jnp.float32),
            pltpu.SemaphoreType.DMA,
        ],
    )(_sc_body)


def kernel(c_indices, d_indices, input, n_voters, num_winners, batch_size,
           num_candidates):
    xt = jnp.transpose(input)
    ct = jnp.transpose(c_indices.astype(jnp.int32))
    dt = jnp.transpose(d_indices.astype(jnp.int32))
    thr = jnp.full((16,), (n_voters // 2 + 1), dtype=jnp.float32)
    acc, dpk = _tc_sweep(xt, xt, ct, dt)
    partials = _sc_final_kernel()(acc, dpk, thr)
    return jnp.sum(partials)
